# baseline (speedup 1.0000x reference)
import jax
import jax.numpy as jnp
from jax import lax
from jax.experimental import pallas as pl
from jax.experimental.pallas import tpu as pltpu

B = 4096
N = 32
VD = 256
MD = 6
WS = 256
MT = 768
HM = 512
GH = 384
WRITE_TH = 0.55
CONTR_TH = 0.6
TEMP = 0.25
DECAY = 0.995

R = 128
NUM = B // R
NB = 3


def _fused(ws_ref, slow_ref, vals_hbm, conf_ref, exp_ref, contr_ref,
           alive_ref, md0_ref, md1_ref, md2_ref, md3_ref, md4_ref, md5_ref,
           w1_0, w1_1, w1_2, w1_3, b1_ref, w2_ref, b2_ref,
           wg_0, wg_1, wg_2, wg_3, wg_4, wg1d_ref, bg1_ref, wg2_ref, bg2_ref,
           wc_0, wc_1, wc_2, wc_3, wc_4, wc1d_ref, bc1_ref, wc2_ref, bc2_ref,
           cand_o, probs_o, vals_out_hbm, conf_o, exp_o, contr_o, alive_o,
           md_o, inbuf, outbuf, in_sem, out_sem):
    i = pl.program_id(0)

    def in_copy(chunk, jj):
        return pltpu.make_async_copy(
            vals_hbm.at[pl.ds(chunk * R, R)], inbuf.at[jj], in_sem.at[jj])

    def out_copy(chunk, jj):
        return pltpu.make_async_copy(
            outbuf.at[jj], vals_out_hbm.at[pl.ds(chunk * R, R)],
            out_sem.at[jj])

    @pl.when(i == 0)
    def _():
        for k in range(NB):
            in_copy(k, k).start()

    bf16 = jnp.bfloat16
    f32 = jnp.float32

    def mm(a, b_ref):
        return jnp.matmul(a, b_ref[...], preferred_element_type=f32)

    ws = ws_ref[...]
    slow = slow_ref[...]
    wsb = ws.astype(bf16)
    slowb = slow.astype(bf16)
    s0 = slowb[:, :256]
    s1 = slowb[:, 256:512]
    s2 = slowb[:, 512:]

    h = (mm(wsb, w1_0) + mm(s0, w1_1) + mm(s1, w1_2) + mm(s2, w1_3)
         + b1_ref[...])
    h = jax.nn.gelu(h)
    cand = mm(h.astype(bf16), w2_ref) + b2_ref[...]
    candb = cand.astype(bf16)

    conf = conf_ref[...]
    alive = alive_ref[...]
    w = alive * conf
    denom = jnp.maximum(jnp.sum(w, axis=1, keepdims=True), 1.0)
    md4 = md4_ref[...]
    md5 = md5_ref[...]
    cols = (md0_ref[...], md1_ref[...], md2_ref[...], md3_ref[...], md4, md5)
    msum8 = jnp.concatenate(
        [jnp.sum(c * w, axis=1, keepdims=True) for c in cols]
        + [jnp.zeros((w.shape[0], 2), w.dtype)], axis=1) / denom
    msum8b = msum8.astype(bf16)

    gpre = (mm(wsb, wg_0) + mm(s0, wg_1) + mm(s1, wg_2) + mm(s2, wg_3)
            + mm(candb, wg_4) + mm(msum8b, wg1d_ref) + bg1_ref[...])
    g = jax.nn.gelu(gpre)
    wp = jax.nn.sigmoid(
        jnp.sum(g * wg2_ref[...], axis=1, keepdims=True) + bg2_ref[...])
    cpre = (mm(wsb, wc_0) + mm(s0, wc_1) + mm(s1, wc_2) + mm(s2, wc_3)
            + mm(candb, wc_4) + mm(msum8b, wc1d_ref) + bc1_ref[...])
    c = jax.nn.gelu(cpre)
    cp = jax.nn.sigmoid(
        jnp.sum(c * wc2_ref[...], axis=1, keepdims=True) + bc2_ref[...])

    wmask = jax.nn.sigmoid((wp - WRITE_TH) / TEMP)
    cmask = jax.nn.sigmoid((cp - CONTR_TH) / TEMP)

    expv = exp_ref[...]
    contr = contr_ref[...]
    inactive = (alive < 0.5)
    has_in = jnp.max(inactive.astype(jnp.float32), axis=1, keepdims=True) > 0.5
    iota = lax.broadcasted_iota(jnp.int32, inactive.shape, 1)
    in_idx = jnp.min(jnp.where(inactive, iota, N), axis=1, keepdims=True)
    util = conf * expv * (1.0 - contr)
    umin = jnp.min(util, axis=1, keepdims=True)
    rep_idx = jnp.min(jnp.where(util == umin, iota, N), axis=1, keepdims=True)
    slot = jnp.where(has_in, in_idx, rep_idx)

    hot = (iota == slot)
    wslot = jnp.where(hot, wmask, 0.0)
    keep = 1.0 - wslot
    conf_n = keep * (conf * DECAY) + wslot * wmask
    exp_n = keep * (expv * DECAY) + wslot
    contr_n = keep * contr + wslot * cmask
    alive_n = jnp.clip(keep * alive + wslot, 0.0, 1.0)
    conf_o[...] = conf_n
    exp_o[...] = exp_n
    contr_o[...] = contr_n
    alive_o[...] = alive_n
    md_o[...] = jnp.concatenate(
        [conf_n, exp_n, contr_n, alive_n, keep * md4, keep * md5], axis=1)

    cand_o[...] = cand
    probs_o[...] = jnp.concatenate([wp, cp, wmask, cmask], axis=1)

    j = lax.rem(i, NB)
    for jj in range(NB):
        @pl.when(j == jj)
        def _(jj=jj):
            in_copy(i, jj).wait()

    jo = lax.rem(i, 2)
    for jj in range(2):
        @pl.when(jnp.logical_and(jo == jj, i >= 2))
        def _(jj=jj):
            out_copy(i - 2, jj).wait()

    vals = inbuf[j]
    iota3 = lax.broadcasted_iota(jnp.int32, (R, N, 1), 1)
    wslot3 = jnp.where(iota3 == slot[:, :, None], wmask[:, :, None], 0.0)
    outbuf[jo] = vals + wslot3 * (cand[:, None, :] - vals)

    for jj in range(2):
        @pl.when(jo == jj)
        def _(jj=jj):
            out_copy(i, jj).start()

    @pl.when(i + NB < NUM)
    def _():
        for jj in range(NB):
            @pl.when(j == jj)
            def _(jj=jj):
                in_copy(i + NB, jj).start()

    @pl.when(i == NUM - 1)
    def _():
        out_copy(NUM - 2, (NUM - 2) % 2).wait()
        out_copy(NUM - 1, (NUM - 1) % 2).wait()


def kernel(workspace, slow_summary_token, state_values, state_confidence,
           state_expiry, state_contradiction, state_alive, state_metadata,
           W1, b1, W2, b2, Wg1, bg1, Wg2, bg2, Wc1, bc1, Wc2, bc2):
    f32 = jnp.float32
    conf2 = state_confidence[..., 0]
    exp2 = state_expiry[..., 0]
    contr2 = state_contradiction[..., 0]
    alive2 = state_alive[..., 0]
    mdcols = tuple(state_metadata[..., j] for j in range(MD))

    bf16 = jnp.bfloat16
    pad2 = jnp.zeros((2, GH), f32)
    wg1d = jnp.concatenate([Wg1[WS + MT + VD:], pad2], axis=0).astype(bf16)
    wc1d = jnp.concatenate([Wc1[WS + MT + VD:], pad2], axis=0).astype(bf16)
    W1b = W1.astype(bf16)
    W2b = W2.astype(bf16)
    Wg1b = Wg1.astype(bf16)
    Wc1b = Wc1.astype(bf16)

    grid = (NUM,)

    def row(shape):
        return pl.BlockSpec((R,) + shape, lambda i: (i,) + (0,) * len(shape))

    def full(shape):
        return pl.BlockSpec(shape, lambda i: (0,) * len(shape))

    def chunk(shape, j):
        return pl.BlockSpec(shape, lambda i, j=j: (j, 0))

    any_spec = pl.BlockSpec(memory_space=pl.ANY)

    in_specs = [
        row((WS,)), row((MT,)), any_spec,
        row((N,)), row((N,)), row((N,)), row((N,)),
        row((N,)), row((N,)), row((N,)), row((N,)), row((N,)), row((N,)),
        chunk((WS, HM), 0), chunk((WS, HM), 1), chunk((WS, HM), 2),
        chunk((WS, HM), 3), full((1, HM)),
        full((HM, VD)), full((1, VD)),
        chunk((WS, GH), 0), chunk((WS, GH), 1), chunk((WS, GH), 2),
        chunk((WS, GH), 3), chunk((WS, GH), 4), full((8, GH)),
        full((1, GH)), full((1, GH)), full((1, 1)),
        chunk((WS, GH), 0), chunk((WS, GH), 1), chunk((WS, GH), 2),
        chunk((WS, GH), 3), chunk((WS, GH), 4), full((8, GH)),
        full((1, GH)), full((1, GH)), full((1, 1)),
    ]
    out_specs = [
        row((VD,)), row((4,)),
        any_spec, row((N,)), row((N,)), row((N,)), row((N,)),
        row((N * MD,)),
    ]
    out_shapes = [
        jax.ShapeDtypeStruct((B, VD), f32),
        jax.ShapeDtypeStruct((B, 4), f32),
        jax.ShapeDtypeStruct((B, N, VD), f32),
        jax.ShapeDtypeStruct((B, N), f32),
        jax.ShapeDtypeStruct((B, N), f32),
        jax.ShapeDtypeStruct((B, N), f32),
        jax.ShapeDtypeStruct((B, N), f32),
        jax.ShapeDtypeStruct((B, N * MD), f32),
    ]
    scratch_shapes = [
        pltpu.MemorySpace.VMEM((NB, R, N, VD), f32),
        pltpu.MemorySpace.VMEM((2, R, N, VD), f32),
        pltpu.SemaphoreType.DMA((NB,)),
        pltpu.SemaphoreType.DMA((2,)),
    ]

    outs = pl.pallas_call(
        _fused,
        grid=grid,
        in_specs=in_specs,
        out_specs=out_specs,
        out_shape=out_shapes,
        scratch_shapes=scratch_shapes,
    )(workspace, slow_summary_token, state_values,
      conf2, exp2, contr2, alive2, *mdcols,
      W1b, W1b, W1b, W1b, b1[None, :], W2b, b2[None, :],
      Wg1b, Wg1b, Wg1b, Wg1b, Wg1b, wg1d, bg1[None, :], Wg2.reshape(1, GH),
      bg2.reshape(1, 1),
      Wc1b, Wc1b, Wc1b, Wc1b, Wc1b, wc1d, bc1[None, :], Wc2.reshape(1, GH),
      bc2.reshape(1, 1))

    (cand, probs, vals, conf_n, exp_n, contr_n, alive_n, md_n) = outs
    metadata = md_n.reshape(B, MD, N).transpose(0, 2, 1)
    return (cand, probs[:, 0:1], probs[:, 1:2], probs[:, 2:3], probs[:, 3:4],
            vals, conf_n[..., None], exp_n[..., None], contr_n[..., None],
            alive_n[..., None], metadata)


# speedup vs baseline: 1.0263x; 1.0263x over previous
import jax
import jax.numpy as jnp
from jax import lax
from jax.experimental import pallas as pl
from jax.experimental.pallas import tpu as pltpu

B = 4096
N = 32
VD = 256
MD = 6
WS = 256
MT = 768
HM = 512
GH = 384
WRITE_TH = 0.55
CONTR_TH = 0.6
TEMP = 0.25
DECAY = 0.995

R = 128
NUM = B // R
NB = 2


def _fused(ws_ref, slow_ref, vals_hbm, conf_ref, exp_ref, contr_ref,
           alive_ref, md0_ref, md1_ref, md2_ref, md3_ref, md4_ref, md5_ref,
           w1_0, w1_1, w1_2, w1_3, b1_ref, w2_ref, b2_ref,
           wg_0, wg_1, wg_2, wg_3, wg_4, wg1d_ref, bg1_ref, wg2_ref, bg2_ref,
           wc_0, wc_1, wc_2, wc_3, wc_4, wc1d_ref, bc1_ref, wc2_ref, bc2_ref,
           cand_o, probs_o, vals_out_hbm, conf_o, exp_o, contr_o, alive_o,
           md_o, inbuf, outbuf, in_sem, out_sem):
    i = pl.program_id(0)

    def in_copy(chunk, jj):
        return pltpu.make_async_copy(
            vals_hbm.at[pl.ds(chunk * R, R)], inbuf.at[jj], in_sem.at[jj])

    def out_copy(chunk, jj):
        return pltpu.make_async_copy(
            outbuf.at[jj], vals_out_hbm.at[pl.ds(chunk * R, R)],
            out_sem.at[jj])

    @pl.when(i == 0)
    def _():
        for k in range(NB):
            in_copy(k, k).start()

    bf16 = jnp.bfloat16
    f32 = jnp.float32

    def mm(a, b_ref):
        return jnp.matmul(a, b_ref[...], preferred_element_type=f32)

    base = i * R
    ws = ws_ref[pl.ds(base, R), :]
    slow = slow_ref[pl.ds(base, R), :]
    wsb = ws.astype(bf16)
    slowb = slow.astype(bf16)
    s0 = slowb[:, :256]
    s1 = slowb[:, 256:512]
    s2 = slowb[:, 512:]

    h = (mm(wsb, w1_0) + mm(s0, w1_1) + mm(s1, w1_2) + mm(s2, w1_3)
         + b1_ref[...])
    h = jax.nn.gelu(h)
    cand = mm(h.astype(bf16), w2_ref) + b2_ref[...]
    candb = cand.astype(bf16)

    conf = conf_ref[pl.ds(base, R), :]
    alive = alive_ref[pl.ds(base, R), :]
    w = alive * conf
    denom = jnp.maximum(jnp.sum(w, axis=1, keepdims=True), 1.0)
    md4 = md4_ref[pl.ds(base, R), :]
    md5 = md5_ref[pl.ds(base, R), :]
    cols = (md0_ref[pl.ds(base, R), :], md1_ref[pl.ds(base, R), :],
            md2_ref[pl.ds(base, R), :], md3_ref[pl.ds(base, R), :], md4, md5)
    msum8 = jnp.concatenate(
        [jnp.sum(c * w, axis=1, keepdims=True) for c in cols]
        + [jnp.zeros((w.shape[0], 2), w.dtype)], axis=1) / denom
    msum8b = msum8.astype(bf16)

    gpre = (mm(wsb, wg_0) + mm(s0, wg_1) + mm(s1, wg_2) + mm(s2, wg_3)
            + mm(candb, wg_4) + mm(msum8b, wg1d_ref) + bg1_ref[...])
    g = jax.nn.gelu(gpre)
    wp = jax.nn.sigmoid(
        jnp.sum(g * wg2_ref[...], axis=1, keepdims=True) + bg2_ref[...])
    cpre = (mm(wsb, wc_0) + mm(s0, wc_1) + mm(s1, wc_2) + mm(s2, wc_3)
            + mm(candb, wc_4) + mm(msum8b, wc1d_ref) + bc1_ref[...])
    c = jax.nn.gelu(cpre)
    cp = jax.nn.sigmoid(
        jnp.sum(c * wc2_ref[...], axis=1, keepdims=True) + bc2_ref[...])

    wmask = jax.nn.sigmoid((wp - WRITE_TH) / TEMP)
    cmask = jax.nn.sigmoid((cp - CONTR_TH) / TEMP)

    expv = exp_ref[pl.ds(base, R), :]
    contr = contr_ref[pl.ds(base, R), :]
    inactive = (alive < 0.5)
    has_in = jnp.max(inactive.astype(jnp.float32), axis=1, keepdims=True) > 0.5
    iota = lax.broadcasted_iota(jnp.int32, inactive.shape, 1)
    in_idx = jnp.min(jnp.where(inactive, iota, N), axis=1, keepdims=True)
    util = conf * expv * (1.0 - contr)
    umin = jnp.min(util, axis=1, keepdims=True)
    rep_idx = jnp.min(jnp.where(util == umin, iota, N), axis=1, keepdims=True)
    slot = jnp.where(has_in, in_idx, rep_idx)

    hot = (iota == slot)
    wslot = jnp.where(hot, wmask, 0.0)
    keep = 1.0 - wslot
    conf_n = keep * (conf * DECAY) + wslot * wmask
    exp_n = keep * (expv * DECAY) + wslot
    contr_n = keep * contr + wslot * cmask
    alive_n = jnp.clip(keep * alive + wslot, 0.0, 1.0)
    conf_o[...] = conf_n
    exp_o[...] = exp_n
    contr_o[...] = contr_n
    alive_o[...] = alive_n
    md_o[...] = jnp.concatenate(
        [conf_n, exp_n, contr_n, alive_n, keep * md4, keep * md5], axis=1)

    cand_o[...] = cand
    probs_o[...] = jnp.concatenate([wp, cp, wmask, cmask], axis=1)

    j = lax.rem(i, NB)
    for jj in range(NB):
        @pl.when(j == jj)
        def _(jj=jj):
            in_copy(i, jj).wait()

    jo = lax.rem(i, 2)
    for jj in range(2):
        @pl.when(jnp.logical_and(jo == jj, i >= 2))
        def _(jj=jj):
            out_copy(i - 2, jj).wait()

    vals = inbuf[j]
    iota3 = lax.broadcasted_iota(jnp.int32, (R, N, 1), 1)
    wslot3 = jnp.where(iota3 == slot[:, :, None], wmask[:, :, None], 0.0)
    outbuf[jo] = vals + wslot3 * (cand[:, None, :] - vals)

    for jj in range(2):
        @pl.when(jo == jj)
        def _(jj=jj):
            out_copy(i, jj).start()

    @pl.when(i + NB < NUM)
    def _():
        for jj in range(NB):
            @pl.when(j == jj)
            def _(jj=jj):
                in_copy(i + NB, jj).start()

    @pl.when(i == NUM - 1)
    def _():
        out_copy(NUM - 2, (NUM - 2) % 2).wait()
        out_copy(NUM - 1, (NUM - 1) % 2).wait()


def kernel(workspace, slow_summary_token, state_values, state_confidence,
           state_expiry, state_contradiction, state_alive, state_metadata,
           W1, b1, W2, b2, Wg1, bg1, Wg2, bg2, Wc1, bc1, Wc2, bc2):
    f32 = jnp.float32
    conf2 = state_confidence[..., 0]
    exp2 = state_expiry[..., 0]
    contr2 = state_contradiction[..., 0]
    alive2 = state_alive[..., 0]
    mdcols = tuple(state_metadata[..., j] for j in range(MD))

    bf16 = jnp.bfloat16
    pad2 = jnp.zeros((2, GH), f32)
    wg1d = jnp.concatenate([Wg1[WS + MT + VD:], pad2], axis=0).astype(bf16)
    wc1d = jnp.concatenate([Wc1[WS + MT + VD:], pad2], axis=0).astype(bf16)
    W1b = W1.astype(bf16)
    W2b = W2.astype(bf16)
    Wg1b = Wg1.astype(bf16)
    Wc1b = Wc1.astype(bf16)

    grid = (NUM,)

    def row(shape):
        return pl.BlockSpec((R,) + shape, lambda i: (i,) + (0,) * len(shape))

    def full(shape):
        return pl.BlockSpec(shape, lambda i: (0,) * len(shape))

    def chunk(shape, j):
        return pl.BlockSpec(shape, lambda i, j=j: (j, 0))

    any_spec = pl.BlockSpec(memory_space=pl.ANY)

    def whole(shape):
        return pl.BlockSpec(shape, lambda i: (0,) * len(shape))

    in_specs = [
        whole((B, WS)), whole((B, MT)), any_spec,
        whole((B, N)), whole((B, N)), whole((B, N)), whole((B, N)),
        whole((B, N)), whole((B, N)), whole((B, N)), whole((B, N)),
        whole((B, N)), whole((B, N)),
        chunk((WS, HM), 0), chunk((WS, HM), 1), chunk((WS, HM), 2),
        chunk((WS, HM), 3), full((1, HM)),
        full((HM, VD)), full((1, VD)),
        chunk((WS, GH), 0), chunk((WS, GH), 1), chunk((WS, GH), 2),
        chunk((WS, GH), 3), chunk((WS, GH), 4), full((8, GH)),
        full((1, GH)), full((1, GH)), full((1, 1)),
        chunk((WS, GH), 0), chunk((WS, GH), 1), chunk((WS, GH), 2),
        chunk((WS, GH), 3), chunk((WS, GH), 4), full((8, GH)),
        full((1, GH)), full((1, GH)), full((1, 1)),
    ]
    out_specs = [
        row((VD,)), row((4,)),
        any_spec, row((N,)), row((N,)), row((N,)), row((N,)),
        row((N * MD,)),
    ]
    out_shapes = [
        jax.ShapeDtypeStruct((B, VD), f32),
        jax.ShapeDtypeStruct((B, 4), f32),
        jax.ShapeDtypeStruct((B, N, VD), f32),
        jax.ShapeDtypeStruct((B, N), f32),
        jax.ShapeDtypeStruct((B, N), f32),
        jax.ShapeDtypeStruct((B, N), f32),
        jax.ShapeDtypeStruct((B, N), f32),
        jax.ShapeDtypeStruct((B, N * MD), f32),
    ]
    scratch_shapes = [
        pltpu.MemorySpace.VMEM((NB, R, N, VD), f32),
        pltpu.MemorySpace.VMEM((2, R, N, VD), f32),
        pltpu.SemaphoreType.DMA((NB,)),
        pltpu.SemaphoreType.DMA((2,)),
    ]

    outs = pl.pallas_call(
        _fused,
        grid=grid,
        in_specs=in_specs,
        out_specs=out_specs,
        out_shape=out_shapes,
        scratch_shapes=scratch_shapes,
    )(workspace, slow_summary_token, state_values,
      conf2, exp2, contr2, alive2, *mdcols,
      W1b, W1b, W1b, W1b, b1[None, :], W2b, b2[None, :],
      Wg1b, Wg1b, Wg1b, Wg1b, Wg1b, wg1d, bg1[None, :], Wg2.reshape(1, GH),
      bg2.reshape(1, 1),
      Wc1b, Wc1b, Wc1b, Wc1b, Wc1b, wc1d, bc1[None, :], Wc2.reshape(1, GH),
      bc2.reshape(1, 1))

    (cand, probs, vals, conf_n, exp_n, contr_n, alive_n, md_n) = outs
    metadata = md_n.reshape(B, MD, N).transpose(0, 2, 1)
    return (cand, probs[:, 0:1], probs[:, 1:2], probs[:, 2:3], probs[:, 3:4],
            vals, conf_n[..., None], exp_n[..., None], contr_n[..., None],
            alive_n[..., None], metadata)


# speedup vs baseline: 1.0485x; 1.0217x over previous
import jax
import jax.numpy as jnp
from jax import lax
from jax.experimental import pallas as pl

B = 4096
N = 32
VD = 256
MD = 6
WS = 256
MT = 768
HM = 512
GH = 384
WRITE_TH = 0.55
CONTR_TH = 0.6
TEMP = 0.25
DECAY = 0.995

R = 256


def _fused(ws_ref, slow_ref, vals_ref, conf_ref, exp_ref, contr_ref,
           alive_ref, md0_ref, md1_ref, md2_ref, md3_ref, md4_ref, md5_ref,
           w1_0, w1_1, w1_2, w1_3, b1_ref, w2_ref, b2_ref,
           wg_0, wg_1, wg_2, wg_3, wg_4, wg1d_ref, bg1_ref, wg2_ref, bg2_ref,
           wc_0, wc_1, wc_2, wc_3, wc_4, wc1d_ref, bc1_ref, wc2_ref, bc2_ref,
           cand_o, probs_o, vals_o, conf_o, exp_o, contr_o, alive_o, md_o):
    ws = ws_ref[...]
    slow = slow_ref[...]
    s0 = slow[:, :256]
    s1 = slow[:, 256:512]
    s2 = slow[:, 512:]

    h = (ws @ w1_0[...] + s0 @ w1_1[...] + s1 @ w1_2[...] + s2 @ w1_3[...]
         + b1_ref[...])
    h = jax.nn.gelu(h)
    cand = h @ w2_ref[...] + b2_ref[...]

    conf = conf_ref[...]
    alive = alive_ref[...]
    w = alive * conf
    denom = jnp.maximum(jnp.sum(w, axis=1, keepdims=True), 1.0)
    md4 = md4_ref[...]
    md5 = md5_ref[...]
    cols = (md0_ref[...], md1_ref[...], md2_ref[...], md3_ref[...], md4, md5)
    msum8 = jnp.concatenate(
        [jnp.sum(c * w, axis=1, keepdims=True) for c in cols]
        + [jnp.zeros((w.shape[0], 2), w.dtype)], axis=1) / denom

    gpre = (ws @ wg_0[...] + s0 @ wg_1[...] + s1 @ wg_2[...] + s2 @ wg_3[...]
            + cand @ wg_4[...] + msum8 @ wg1d_ref[...] + bg1_ref[...])
    g = jax.nn.gelu(gpre)
    wp = jax.nn.sigmoid(
        jnp.sum(g * wg2_ref[...], axis=1, keepdims=True) + bg2_ref[...])
    cpre = (ws @ wc_0[...] + s0 @ wc_1[...] + s1 @ wc_2[...] + s2 @ wc_3[...]
            + cand @ wc_4[...] + msum8 @ wc1d_ref[...] + bc1_ref[...])
    c = jax.nn.gelu(cpre)
    cp = jax.nn.sigmoid(
        jnp.sum(c * wc2_ref[...], axis=1, keepdims=True) + bc2_ref[...])

    wmask = jax.nn.sigmoid((wp - WRITE_TH) / TEMP)
    cmask = jax.nn.sigmoid((cp - CONTR_TH) / TEMP)

    expv = exp_ref[...]
    contr = contr_ref[...]
    inactive = (alive < 0.5)
    has_in = jnp.max(inactive.astype(jnp.float32), axis=1, keepdims=True) > 0.5
    iota = lax.broadcasted_iota(jnp.int32, inactive.shape, 1)
    in_idx = jnp.min(jnp.where(inactive, iota, N), axis=1, keepdims=True)
    util = conf * expv * (1.0 - contr)
    umin = jnp.min(util, axis=1, keepdims=True)
    rep_idx = jnp.min(jnp.where(util == umin, iota, N), axis=1, keepdims=True)
    slot = jnp.where(has_in, in_idx, rep_idx)

    hot = (iota == slot)
    wslot = jnp.where(hot, wmask, 0.0)
    keep = 1.0 - wslot
    conf_n = keep * (conf * DECAY) + wslot * wmask
    exp_n = keep * (expv * DECAY) + wslot
    contr_n = keep * contr + wslot * cmask
    alive_n = jnp.clip(keep * alive + wslot, 0.0, 1.0)
    conf_o[...] = conf_n
    exp_o[...] = exp_n
    contr_o[...] = contr_n
    alive_o[...] = alive_n
    md_o[...] = jnp.concatenate(
        [conf_n, exp_n, contr_n, alive_n, keep * md4, keep * md5], axis=1)

    vals = vals_ref[...]
    iota3 = lax.broadcasted_iota(jnp.int32, (vals.shape[0], N, 1), 1)
    wslot3 = jnp.where(iota3 == slot[:, :, None], wmask[:, :, None], 0.0)
    vals_o[...] = vals + wslot3 * (cand[:, None, :] - vals)

    cand_o[...] = cand
    probs_o[...] = jnp.concatenate([wp, cp, wmask, cmask], axis=1)


def kernel(workspace, slow_summary_token, state_values, state_confidence,
           state_expiry, state_contradiction, state_alive, state_metadata,
           W1, b1, W2, b2, Wg1, bg1, Wg2, bg2, Wc1, bc1, Wc2, bc2):
    f32 = jnp.float32
    conf2 = state_confidence[..., 0]
    exp2 = state_expiry[..., 0]
    contr2 = state_contradiction[..., 0]
    alive2 = state_alive[..., 0]
    mdcols = tuple(state_metadata[..., j] for j in range(MD))

    pad2 = jnp.zeros((2, GH), f32)
    wg1d = jnp.concatenate([Wg1[WS + MT + VD:], pad2], axis=0)
    wc1d = jnp.concatenate([Wc1[WS + MT + VD:], pad2], axis=0)

    grid = (B // R,)

    def row(shape):
        return pl.BlockSpec((R,) + shape, lambda i: (i,) + (0,) * len(shape))

    def full(shape):
        return pl.BlockSpec(shape, lambda i: (0,) * len(shape))

    def chunk(shape, j):
        return pl.BlockSpec(shape, lambda i, j=j: (j, 0))

    in_specs = [
        row((WS,)), row((MT,)), row((N, VD)),
        row((N,)), row((N,)), row((N,)), row((N,)),
        row((N,)), row((N,)), row((N,)), row((N,)), row((N,)), row((N,)),
        chunk((WS, HM), 0), chunk((WS, HM), 1), chunk((WS, HM), 2),
        chunk((WS, HM), 3), full((1, HM)),
        full((HM, VD)), full((1, VD)),
        chunk((WS, GH), 0), chunk((WS, GH), 1), chunk((WS, GH), 2),
        chunk((WS, GH), 3), chunk((WS, GH), 4), full((8, GH)),
        full((1, GH)), full((1, GH)), full((1, 1)),
        chunk((WS, GH), 0), chunk((WS, GH), 1), chunk((WS, GH), 2),
        chunk((WS, GH), 3), chunk((WS, GH), 4), full((8, GH)),
        full((1, GH)), full((1, GH)), full((1, 1)),
    ]
    out_specs = [
        row((VD,)), row((4,)),
        row((N, VD)), row((N,)), row((N,)), row((N,)), row((N,)),
        row((N * MD,)),
    ]
    out_shapes = [
        jax.ShapeDtypeStruct((B, VD), f32),
        jax.ShapeDtypeStruct((B, 4), f32),
        jax.ShapeDtypeStruct((B, N, VD), f32),
        jax.ShapeDtypeStruct((B, N), f32),
        jax.ShapeDtypeStruct((B, N), f32),
        jax.ShapeDtypeStruct((B, N), f32),
        jax.ShapeDtypeStruct((B, N), f32),
        jax.ShapeDtypeStruct((B, N * MD), f32),
    ]

    outs = pl.pallas_call(
        _fused,
        grid=grid,
        in_specs=in_specs,
        out_specs=out_specs,
        out_shape=out_shapes,
    )(workspace, slow_summary_token, state_values,
      conf2, exp2, contr2, alive2, *mdcols,
      W1, W1, W1, W1, b1[None, :], W2, b2[None, :],
      Wg1, Wg1, Wg1, Wg1, Wg1, wg1d, bg1[None, :], Wg2.reshape(1, GH),
      bg2.reshape(1, 1),
      Wc1, Wc1, Wc1, Wc1, Wc1, wc1d, bc1[None, :], Wc2.reshape(1, GH),
      bc2.reshape(1, 1))

    (cand, probs, vals, conf_n, exp_n, contr_n, alive_n, md_n) = outs
    metadata = md_n.reshape(B, MD, N).transpose(0, 2, 1)
    return (cand, probs[:, 0:1], probs[:, 1:2], probs[:, 2:3], probs[:, 3:4],
            vals, conf_n[..., None], exp_n[..., None], contr_n[..., None],
            alive_n[..., None], metadata)
